# Initial kernel scaffold; baseline (speedup 1.0000x reference)
import jax
import jax.numpy as jnp
from jax import lax
from jax.experimental import pallas as pl
from jax.experimental.pallas import tpu as pltpu

_N = 20000
_NUM = 2048
_ROWS = 160
_LANES = 128
_BIG = jnp.int32(2 ** 30)


def _fps_body(x_ref, y_ref, z_ref, w_ref, out_ref, dm_ref):
    ri = lax.broadcasted_iota(jnp.int32, (_ROWS, _LANES), 0)
    ci = lax.broadcasted_iota(jnp.int32, (_ROWS, _LANES), 1)
    gidx = ri * _LANES + ci
    valid = gidx < _N

    inf = jnp.float32(jnp.inf)
    dm_ref[:] = jnp.where(valid, inf, -inf)

    x = x_ref[:]
    y = y_ref[:]
    z = z_ref[:]

    lane_io = lax.broadcasted_iota(jnp.int32, (1, _LANES), 1)

    def extract(ref, r, c):
        rowv = ref[pl.ds(r, 1), :]
        return jnp.sum(jnp.where(lane_io == c, rowv, 0.0))

    def body(i, carry):
        cx, cy, cz, cw = carry
        rowv = jnp.where(
            lane_io == 0, cx,
            jnp.where(lane_io == 1, cy, jnp.where(lane_io == 2, cz, cw)))
        out_ref[pl.ds(i, 1), :] = rowv
        dx = x - cx
        dy = y - cy
        dz = z - cz
        d = (dx * dx + dy * dy) + dz * dz
        dm = jnp.minimum(dm_ref[:], d)
        dm_ref[:] = dm
        mx = jnp.max(dm)
        sel = jnp.min(jnp.where(dm == mx, gidx, _BIG))
        r = sel // _LANES
        c = sel % _LANES
        return (extract(x_ref, r, c), extract(y_ref, r, c),
                extract(z_ref, r, c), extract(w_ref, r, c))

    init = (x_ref[0, 0], y_ref[0, 0], z_ref[0, 0], w_ref[0, 0])
    lax.fori_loop(0, _NUM, body, init)


def kernel(pcd):
    xyz = pcd[:, :3]
    xyz = xyz - jnp.mean(xyz, axis=0, keepdims=True)
    dis = jnp.linalg.norm(xyz, axis=1)
    max_dis = jnp.max(dis)
    xyz = xyz / max_dis
    pcdn = pcd.at[:, :3].set(xyz)

    pad = jnp.zeros((_ROWS * _LANES - _N, 4), pcdn.dtype)
    p = jnp.concatenate([pcdn, pad], axis=0)
    xp = p[:, 0].reshape(_ROWS, _LANES)
    yp = p[:, 1].reshape(_ROWS, _LANES)
    zp = p[:, 2].reshape(_ROWS, _LANES)
    wp = p[:, 3].reshape(_ROWS, _LANES)

    res = pl.pallas_call(
        _fps_body,
        out_shape=jax.ShapeDtypeStruct((_NUM, _LANES), jnp.float32),
        scratch_shapes=[pltpu.VMEM((_ROWS, _LANES), jnp.float32)],
    )(xp, yp, zp, wp)
    return res[:, :4].T


# speedup vs baseline: 27.0966x; 27.0966x over previous
import jax
import jax.numpy as jnp
from jax import lax
from jax.experimental import pallas as pl
from jax.experimental.pallas import tpu as pltpu

_N = 20000
_NUM = 2048
_ROWS = 160
_LANES = 128
_BIG = 2 ** 30


def _fps_body(x_ref, y_ref, z_ref, w_ref, out_ref, dm_ref):
    ri = lax.broadcasted_iota(jnp.int32, (_ROWS, _LANES), 0)
    ci = lax.broadcasted_iota(jnp.int32, (_ROWS, _LANES), 1)
    gidx = ri * _LANES + ci
    valid = gidx < _N

    inf = jnp.float32(jnp.inf)
    dm_ref[:] = jnp.where(valid, inf, -inf)

    x = x_ref[:]
    y = y_ref[:]
    z = z_ref[:]

    lane_io = lax.broadcasted_iota(jnp.int32, (1, _LANES), 1)

    def extract(ref, r, c):
        rowv = ref[pl.ds(r, 1), :]
        return jnp.sum(jnp.where(lane_io == c, rowv, 0.0))

    def body(i, carry):
        cx, cy, cz, cw = carry
        rowv = jnp.where(
            lane_io == 0, cx,
            jnp.where(lane_io == 1, cy, jnp.where(lane_io == 2, cz, cw)))
        out_ref[pl.ds(i, 1), :] = rowv
        dx = x - cx
        dy = y - cy
        dz = z - cz
        d = (dx * dx + dy * dy) + dz * dz
        dm = jnp.minimum(dm_ref[:], d)
        dm_ref[:] = dm
        mx = jnp.max(dm)
        sel = jnp.min(jnp.where(dm == mx, gidx, _BIG))
        r = sel // _LANES
        c = sel % _LANES
        return (extract(x_ref, r, c), extract(y_ref, r, c),
                extract(z_ref, r, c), extract(w_ref, r, c))

    init = (x_ref[0, 0], y_ref[0, 0], z_ref[0, 0], w_ref[0, 0])
    lax.fori_loop(0, _NUM, body, init)


def kernel(pcd):
    xyz = pcd[:, :3]
    xyz = xyz - jnp.mean(xyz, axis=0, keepdims=True)
    dis = jnp.linalg.norm(xyz, axis=1)
    max_dis = jnp.max(dis)
    xyz = xyz / max_dis
    pcdn = pcd.at[:, :3].set(xyz)

    pad = jnp.zeros((_ROWS * _LANES - _N, 4), pcdn.dtype)
    p = jnp.concatenate([pcdn, pad], axis=0)
    xp = p[:, 0].reshape(_ROWS, _LANES)
    yp = p[:, 1].reshape(_ROWS, _LANES)
    zp = p[:, 2].reshape(_ROWS, _LANES)
    wp = p[:, 3].reshape(_ROWS, _LANES)

    res = pl.pallas_call(
        _fps_body,
        out_shape=jax.ShapeDtypeStruct((_NUM, _LANES), jnp.float32),
        scratch_shapes=[pltpu.VMEM((_ROWS, _LANES), jnp.float32)],
    )(xp, yp, zp, wp)
    return res[:, :4].T
